# Initial kernel scaffold; baseline (speedup 1.0000x reference)
import jax
import jax.numpy as jnp
from jax.experimental import pallas as pl


def kernel(agent_positions, goal_positions, radius_table, angle_table, W, b):
    raise NotImplementedError("write your pallas kernel here")


# speedup vs baseline: 4.6659x; 4.6659x over previous
import math

import jax
import jax.numpy as jnp
from jax.experimental import pallas as pl
from jax.experimental.pallas import tpu as pltpu

_B = 16384
_BS = 2048
_RV = 512
_AV = 49
_AVP = 64
_ED = 32
_NA = 6


def _body(apT_ref, gpT_ref, rt_ref, at_ref, W_ref, b_ref, out_ref):
    ax = apT_ref[0:1, :]
    az = apT_ref[1:2, :]
    pose = apT_ref[2:3, :]
    gx = gpT_ref[0:1, :]
    gz = gpT_ref[1:2, :]
    dx = gx - ax
    dz = gz - az

    radius = jnp.sqrt(dx * dx + dz * dz)
    r_idx = (radius / 5.0).astype(jnp.int32)

    ang = 90.0 - jnp.arctan2(dz, dx) * (180.0 / math.pi)
    diff = jnp.mod(ang - pose, 360.0)
    t_idx = (diff / 7.5).astype(jnp.int32)

    iota_r = jax.lax.broadcasted_iota(jnp.int32, (_RV, _BS), 0)
    oh_r = (iota_r == r_idx).astype(jnp.float32)
    iota_a = jax.lax.broadcasted_iota(jnp.int32, (_AVP, _BS), 0)
    oh_a = (iota_a == t_idx).astype(jnp.float32)

    Wr = W_ref[:, 0:_ED]
    Wa = W_ref[:, _ED:2 * _ED]
    proj_r = jax.lax.dot_general(rt_ref[...], Wr, (((1,), (1,)), ((), ())),
                                 preferred_element_type=jnp.float32)
    proj_a = jax.lax.dot_general(at_ref[...], Wa, (((1,), (1,)), ((), ())),
                                 preferred_element_type=jnp.float32)

    logits = (
        jax.lax.dot_general(oh_r, proj_r, (((0,), (0,)), ((), ())),
                            preferred_element_type=jnp.float32)
        + jax.lax.dot_general(oh_a, proj_a, (((0,), (0,)), ((), ())),
                              preferred_element_type=jnp.float32)
        + b_ref[...]
    )

    m = jnp.max(logits, axis=1, keepdims=True)
    s = logits - m
    lse = jnp.log(jnp.sum(jnp.exp(s), axis=1, keepdims=True))
    out_ref[...] = s - lse


def kernel(agent_positions, goal_positions, radius_table, angle_table, W, b):
    apT = agent_positions.T
    gpT = goal_positions.T
    at_p = jnp.pad(angle_table, ((0, _AVP - _AV), (0, 0)))
    b2 = b.reshape(1, _NA)
    grid = _B // _BS
    return pl.pallas_call(
        _body,
        grid=(grid,),
        in_specs=[
            pl.BlockSpec((3, _BS), lambda i: (0, i)),
            pl.BlockSpec((2, _BS), lambda i: (0, i)),
            pl.BlockSpec((_RV, _ED), lambda i: (0, 0)),
            pl.BlockSpec((_AVP, _ED), lambda i: (0, 0)),
            pl.BlockSpec((_NA, 2 * _ED), lambda i: (0, 0)),
            pl.BlockSpec((1, _NA), lambda i: (0, 0)),
        ],
        out_specs=pl.BlockSpec((_BS, _NA), lambda i: (i, 0)),
        out_shape=jax.ShapeDtypeStruct((_B, _NA), jnp.float32),
    )(apT, gpT, radius_table, at_p, W, b2)
